# baseline (speedup 1.0000x reference)
import functools

import jax
import jax.numpy as jnp
from jax import lax
from jax.experimental import pallas as pl
from jax.experimental.pallas import tpu as pltpu
from jax.experimental.pallas import tpu_sc as plsc

_NC = 2
_NS = 16
_L = 16
_NW = _NC * _NS

_B = 4096
_D = 64
_BPW = _B // _NW

_mesh = plsc.VectorSubcoreMesh(core_axis_name="c", subcore_axis_name="s")


@functools.partial(
    pl.kernel,
    out_type=(
        jax.ShapeDtypeStruct((_B, _D), jnp.float32),
        jax.ShapeDtypeStruct((_B, _D), jnp.float32),
        jax.ShapeDtypeStruct((_B, _D), jnp.float32),
    ),
    mesh=_mesh,
    compiler_params=pltpu.CompilerParams(use_tc_tiling_on_sc=False),
    scratch_types=[
        pltpu.VMEM((_BPW,), jnp.int32),
        pltpu.VMEM((_BPW,), jnp.int32),
        pltpu.VMEM((_BPW, _D), jnp.float32),
        pltpu.VMEM((_BPW, _D), jnp.float32),
        pltpu.VMEM((_BPW, _D), jnp.float32),
        pltpu.VMEM((_BPW, _D), jnp.float32),
        pltpu.SemaphoreType.DMA,
        pltpu.SemaphoreType.DMA,
        pltpu.SemaphoreType.DMA,
        pltpu.SemaphoreType.DMA,
    ],
)
def _sc_gather(user_hbm, item_hbm, gut_hbm, git_hbm, mut_hbm, mit_hbm,
               gmf_out, mlpu_out, mlpi_out,
               idx_u, idx_i, gu, gi, mu, mi, s0, s1, s2, s3):
    wid = lax.axis_index("s") * _NC + lax.axis_index("c")
    base = wid * _BPW
    pltpu.sync_copy(user_hbm.at[pl.ds(base, _BPW)], idx_u)
    pltpu.sync_copy(item_hbm.at[pl.ds(base, _BPW)], idx_i)
    c0 = pltpu.async_copy(gut_hbm.at[idx_u], gu, s0)
    c1 = pltpu.async_copy(git_hbm.at[idx_i], gi, s1)
    c2 = pltpu.async_copy(mut_hbm.at[idx_u], mu, s2)
    c3 = pltpu.async_copy(mit_hbm.at[idx_i], mi, s3)
    c2.wait()
    pltpu.sync_copy(mu, mlpu_out.at[pl.ds(base, _BPW)])
    c3.wait()
    pltpu.sync_copy(mi, mlpi_out.at[pl.ds(base, _BPW)])
    c0.wait()
    c1.wait()

    def body(r, carry):
        for c in range(_D // _L):
            sl = pl.ds(c * _L, _L)
            gu[r, sl] = gu[r, sl] * gi[r, sl]
        return carry

    lax.fori_loop(0, _BPW, body, 0)
    pltpu.sync_copy(gu, gmf_out.at[pl.ds(base, _BPW)])


def _tc_body(gmf_ref, mu_ref, mi_ref, w1u_ref, w1i_ref, b1_ref,
             w2_ref, b2_ref, wpg_ref, wpm_ref, out_ref):
    h = jnp.dot(mu_ref[...], w1u_ref[...], preferred_element_type=jnp.float32)
    h = h + jnp.dot(mi_ref[...], w1i_ref[...], preferred_element_type=jnp.float32)
    h = jnp.maximum(h + b1_ref[...], 0.0)
    m = jnp.dot(h, w2_ref[...], preferred_element_type=jnp.float32)
    m = jnp.maximum(m + b2_ref[...], 0.0)
    out = jnp.sum(gmf_ref[...] * wpg_ref[...], axis=1, keepdims=True)
    out = out + jnp.sum(m * wpm_ref[...], axis=1, keepdims=True)
    out_ref[...] = out


def kernel(user, item, gmf_user_table, gmf_item_table,
           mlp_user_table, mlp_item_table, W1, b1, W2, b2, Wp):
    user = user.astype(jnp.int32)
    item = item.astype(jnp.int32)
    gmf, mu, mi = _sc_gather(user, item, gmf_user_table, gmf_item_table,
                             mlp_user_table, mlp_item_table)
    out = pl.pallas_call(
        _tc_body,
        out_shape=jax.ShapeDtypeStruct((_B, 1), jnp.float32),
    )(gmf, mu, mi,
      W1[:_D], W1[_D:], b1.reshape(1, _D),
      W2, b2.reshape(1, 32),
      Wp[:_D].reshape(1, _D), Wp[_D:].reshape(1, 32))
    return out.reshape(-1)


# speedup vs baseline: 1.0004x; 1.0004x over previous
import functools

import jax
import jax.numpy as jnp
from jax import lax
from jax.experimental import pallas as pl
from jax.experimental.pallas import tpu as pltpu
from jax.experimental.pallas import tpu_sc as plsc

_NC = 2
_NS = 16
_L = 16
_NW = _NC * _NS

_B = 4096
_D = 64
_BPW = _B // _NW

_mesh = plsc.VectorSubcoreMesh(core_axis_name="c", subcore_axis_name="s")


@functools.partial(
    pl.kernel,
    out_type=(
        jax.ShapeDtypeStruct((_B, 2 * _D), jnp.float32),
        jax.ShapeDtypeStruct((_B, 2 * _D), jnp.float32),
        jax.ShapeDtypeStruct((_B, 2 * _D), jnp.float32),
        jax.ShapeDtypeStruct((_B, 2 * _D), jnp.float32),
    ),
    mesh=_mesh,
    scratch_types=[
        pltpu.VMEM((_BPW,), jnp.int32),
        pltpu.VMEM((_BPW,), jnp.int32),
        pltpu.VMEM((_BPW, 2 * _D), jnp.float32),
        pltpu.VMEM((_BPW, 2 * _D), jnp.float32),
        pltpu.VMEM((_BPW, 2 * _D), jnp.float32),
        pltpu.VMEM((_BPW, 2 * _D), jnp.float32),
        pltpu.SemaphoreType.DMA,
        pltpu.SemaphoreType.DMA,
        pltpu.SemaphoreType.DMA,
        pltpu.SemaphoreType.DMA,
    ],
)
def _sc_gather(user_hbm, item_hbm, gut2, git2, mut2, mit2,
               gu_out, gi_out, mu_out, mi_out,
               idx_u, idx_i, bgu, bgi, bmu, bmi, s0, s1, s2, s3):
    wid = lax.axis_index("s") * _NC + lax.axis_index("c")
    base = wid * _BPW
    pltpu.sync_copy(user_hbm.at[pl.ds(base, _BPW)], idx_u)
    pltpu.sync_copy(item_hbm.at[pl.ds(base, _BPW)], idx_i)
    for k in range(_BPW // _L):
        sl = pl.ds(k * _L, _L)
        idx_u[sl] = idx_u[sl] >> 1
        idx_i[sl] = idx_i[sl] >> 1
    c0 = pltpu.async_copy(gut2.at[idx_u], bgu, s0)
    c1 = pltpu.async_copy(git2.at[idx_i], bgi, s1)
    c2 = pltpu.async_copy(mut2.at[idx_u], bmu, s2)
    c3 = pltpu.async_copy(mit2.at[idx_i], bmi, s3)
    c0.wait()
    pltpu.sync_copy(bgu, gu_out.at[pl.ds(base, _BPW)])
    c1.wait()
    pltpu.sync_copy(bgi, gi_out.at[pl.ds(base, _BPW)])
    c2.wait()
    pltpu.sync_copy(bmu, mu_out.at[pl.ds(base, _BPW)])
    c3.wait()
    pltpu.sync_copy(bmi, mi_out.at[pl.ds(base, _BPW)])


def _tc_body(user_ref, item_ref, gu_ref, gi_ref, mu_ref, mi_ref,
             w1u_ref, w1i_ref, b1_ref, w2_ref, b2_ref, wpg_ref, wpm_ref,
             out_ref):
    pu = (user_ref[...] & 1) == 1
    pi = (item_ref[...] & 1) == 1
    gmf = (jnp.where(pu, gu_ref[:, _D:], gu_ref[:, :_D])
           * jnp.where(pi, gi_ref[:, _D:], gi_ref[:, :_D]))
    mu = jnp.where(pu, mu_ref[:, _D:], mu_ref[:, :_D])
    mi = jnp.where(pi, mi_ref[:, _D:], mi_ref[:, :_D])
    h = jnp.dot(mu, w1u_ref[...], preferred_element_type=jnp.float32)
    h = h + jnp.dot(mi, w1i_ref[...], preferred_element_type=jnp.float32)
    h = jnp.maximum(h + b1_ref[...], 0.0)
    m = jnp.dot(h, w2_ref[...], preferred_element_type=jnp.float32)
    m = jnp.maximum(m + b2_ref[...], 0.0)
    out = jnp.sum(gmf * wpg_ref[...], axis=1, keepdims=True)
    out = out + jnp.sum(m * wpm_ref[...], axis=1, keepdims=True)
    out_ref[...] = out


def kernel(user, item, gmf_user_table, gmf_item_table,
           mlp_user_table, mlp_item_table, W1, b1, W2, b2, Wp):
    user = user.astype(jnp.int32)
    item = item.astype(jnp.int32)
    gu2, gi2, mu2, mi2 = _sc_gather(
        user, item,
        gmf_user_table.reshape(-1, 2 * _D), gmf_item_table.reshape(-1, 2 * _D),
        mlp_user_table.reshape(-1, 2 * _D), mlp_item_table.reshape(-1, 2 * _D))
    out = pl.pallas_call(
        _tc_body,
        out_shape=jax.ShapeDtypeStruct((_B, 1), jnp.float32),
    )(user.reshape(_B, 1), item.reshape(_B, 1),
      gu2, gi2, mu2, mi2,
      W1[:_D], W1[_D:], b1.reshape(1, _D),
      W2, b2.reshape(1, 32),
      Wp[:_D].reshape(1, _D), Wp[_D:].reshape(1, 32))
    return out.reshape(-1)


# speedup vs baseline: 2.9692x; 2.9681x over previous
import functools

import jax
import jax.numpy as jnp
from jax import lax
from jax.experimental import pallas as pl
from jax.experimental.pallas import tpu as pltpu
from jax.experimental.pallas import tpu_sc as plsc

_NC = 2
_NS = 16
_L = 16
_NW = _NC * _NS

_B = 4096
_D = 64
_V = 100000
_TASKS_PER_W = (4 * _D) // _NW

_mesh = plsc.VectorSubcoreMesh(core_axis_name="c", subcore_axis_name="s")


@functools.partial(
    pl.kernel,
    out_type=jax.ShapeDtypeStruct((4 * _D, _B), jnp.float32),
    mesh=_mesh,
    compiler_params=pltpu.CompilerParams(needs_layout_passes=False),
    scratch_types=[
        pltpu.VMEM((2 * _B,), jnp.int32),
        pltpu.VMEM((_V,), jnp.float32),
        pltpu.VMEM((_B,), jnp.float32),
        pltpu.SemaphoreType.DMA,
    ],
)
def _sc_sweep(user_hbm, item_hbm, gutT, gitT, mutT, mitT, out_hbm,
              idx2, rowbuf, outbuf, sem):
    wid = lax.axis_index("s") * _NC + lax.axis_index("c")
    tbl = wid // 8
    dbase = (wid % 8) * _TASKS_PER_W
    pltpu.sync_copy(user_hbm, idx2.at[pl.ds(0, _B)])
    pltpu.sync_copy(item_hbm, idx2.at[pl.ds(_B, _B)])
    ibase = (tbl % 2) * _B

    for k in range(_TASKS_PER_W):
        d = dbase + k
        for t, tab in enumerate((gutT, gitT, mutT, mitT)):
            @pl.when(tbl == t)
            def _():
                pltpu.sync_copy(tab.at[d], rowbuf)

        def chunk(c, carry):
            ids = idx2[pl.ds(ibase + c * _L, _L)]
            outbuf[pl.ds(c * _L, _L)] = plsc.load_gather(rowbuf, [ids])
            return carry

        lax.fori_loop(0, _B // _L, chunk, 0)
        pltpu.sync_copy(outbuf, out_hbm.at[wid * _TASKS_PER_W + k])


def _tc_body(act_ref, w1ut_ref, w1it_ref, b1_ref, w2t_ref, b2_ref,
             wpg_ref, wpm_ref, out_ref):
    gmf = act_ref[0:_D, :] * act_ref[_D:2 * _D, :]
    h = jnp.dot(w1ut_ref[...], act_ref[2 * _D:3 * _D, :],
                preferred_element_type=jnp.float32)
    h = h + jnp.dot(w1it_ref[...], act_ref[3 * _D:4 * _D, :],
                    preferred_element_type=jnp.float32)
    h = jnp.maximum(h + b1_ref[...], 0.0)
    m = jnp.dot(w2t_ref[...], h, preferred_element_type=jnp.float32)
    m = jnp.maximum(m + b2_ref[...], 0.0)
    out = jnp.dot(wpg_ref[...], gmf, preferred_element_type=jnp.float32)
    out = out + jnp.dot(wpm_ref[...], m, preferred_element_type=jnp.float32)
    out_ref[...] = out


def kernel(user, item, gmf_user_table, gmf_item_table,
           mlp_user_table, mlp_item_table, W1, b1, W2, b2, Wp):
    user = user.astype(jnp.int32)
    item = item.astype(jnp.int32)
    act = _sc_sweep(user, item,
                    gmf_user_table.T, gmf_item_table.T,
                    mlp_user_table.T, mlp_item_table.T)
    out = pl.pallas_call(
        _tc_body,
        out_shape=jax.ShapeDtypeStruct((1, _B), jnp.float32),
    )(act,
      W1[:_D].T, W1[_D:].T, b1.reshape(_D, 1),
      W2.T, b2.reshape(32, 1),
      Wp[:_D].reshape(1, _D), Wp[_D:].reshape(1, 32))
    return out.reshape(-1)


# speedup vs baseline: 3.0211x; 1.0175x over previous
import functools

import jax
import jax.numpy as jnp
from jax import lax
from jax.experimental import pallas as pl
from jax.experimental.pallas import tpu as pltpu
from jax.experimental.pallas import tpu_sc as plsc

_NC = 2
_NS = 16
_L = 16
_NW = _NC * _NS

_B = 4096
_D = 64
_V = 100000
_TASKS_PER_W = (4 * _D) // _NW
_UNROLL = 8

_mesh = plsc.VectorSubcoreMesh(core_axis_name="c", subcore_axis_name="s")


@functools.partial(
    pl.kernel,
    out_type=jax.ShapeDtypeStruct((4 * _D, _B), jnp.float32),
    mesh=_mesh,
    compiler_params=pltpu.CompilerParams(needs_layout_passes=False),
    scratch_types=[
        pltpu.VMEM((2 * _B,), jnp.int32),
        pltpu.VMEM((_V,), jnp.float32),
        pltpu.VMEM((_B,), jnp.float32),
        pltpu.SemaphoreType.DMA,
    ],
)
def _sc_sweep(user_hbm, item_hbm, gutT, gitT, mutT, mitT, out_hbm,
              idx2, rowbuf, outbuf, sem):
    wid = lax.axis_index("s") * _NC + lax.axis_index("c")
    tbl = wid // 8
    dbase = (wid % 8) * _TASKS_PER_W
    pltpu.sync_copy(user_hbm, idx2.at[pl.ds(0, _B)])
    pltpu.sync_copy(item_hbm, idx2.at[pl.ds(_B, _B)])
    ibase = (tbl % 2) * _B

    for k in range(_TASKS_PER_W):
        d = dbase + k
        for t, tab in enumerate((gutT, gitT, mutT, mitT)):
            @pl.when(tbl == t)
            def _():
                pltpu.sync_copy(tab.at[d], rowbuf)

        def chunk(c, carry):
            for j in range(_UNROLL):
                off = (c * _UNROLL + j) * _L
                ids = idx2[pl.ds(ibase + off, _L)]
                outbuf[pl.ds(off, _L)] = plsc.load_gather(rowbuf, [ids])
            return carry

        lax.fori_loop(0, _B // (_L * _UNROLL), chunk, 0)
        pltpu.sync_copy(outbuf, out_hbm.at[wid * _TASKS_PER_W + k])


def _tc_body(act_ref, w1ut_ref, w1it_ref, b1_ref, w2t_ref, b2_ref,
             wpg_ref, wpm_ref, out_ref):
    gmf = act_ref[0:_D, :] * act_ref[_D:2 * _D, :]
    h = jnp.dot(w1ut_ref[...], act_ref[2 * _D:3 * _D, :],
                preferred_element_type=jnp.float32)
    h = h + jnp.dot(w1it_ref[...], act_ref[3 * _D:4 * _D, :],
                    preferred_element_type=jnp.float32)
    h = jnp.maximum(h + b1_ref[...], 0.0)
    m = jnp.dot(w2t_ref[...], h, preferred_element_type=jnp.float32)
    m = jnp.maximum(m + b2_ref[...], 0.0)
    out = jnp.dot(wpg_ref[...], gmf, preferred_element_type=jnp.float32)
    out = out + jnp.dot(wpm_ref[...], m, preferred_element_type=jnp.float32)
    out_ref[...] = out


def kernel(user, item, gmf_user_table, gmf_item_table,
           mlp_user_table, mlp_item_table, W1, b1, W2, b2, Wp):
    user = user.astype(jnp.int32)
    item = item.astype(jnp.int32)
    act = _sc_sweep(user, item,
                    gmf_user_table.T, gmf_item_table.T,
                    mlp_user_table.T, mlp_item_table.T)
    out = pl.pallas_call(
        _tc_body,
        out_shape=jax.ShapeDtypeStruct((1, _B), jnp.float32),
    )(act,
      W1[:_D].T, W1[_D:].T, b1.reshape(_D, 1),
      W2.T, b2.reshape(32, 1),
      Wp[:_D].reshape(1, _D), Wp[_D:].reshape(1, 32))
    return out.reshape(-1)


# speedup vs baseline: 3.0563x; 1.0117x over previous
import functools

import jax
import jax.numpy as jnp
from jax import lax
from jax.experimental import pallas as pl
from jax.experimental.pallas import tpu as pltpu
from jax.experimental.pallas import tpu_sc as plsc

_NC = 2
_NS = 16
_L = 16
_NW = _NC * _NS

_B = 4096
_D = 64
_V = 100000
_TASKS_PER_W = (4 * _D) // _NW
_UNROLL = 8

_mesh = plsc.VectorSubcoreMesh(core_axis_name="c", subcore_axis_name="s")


@functools.partial(
    pl.kernel,
    out_type=jax.ShapeDtypeStruct((4 * _D, _B), jnp.float32),
    mesh=_mesh,
    compiler_params=pltpu.CompilerParams(needs_layout_passes=False),
    scratch_types=[
        pltpu.VMEM((2 * _B,), jnp.int32),
        pltpu.VMEM((_V,), jnp.float32),
        pltpu.VMEM((_B,), jnp.float32),
        pltpu.SemaphoreType.DMA,
        pltpu.SemaphoreType.DMA,
        pltpu.SemaphoreType.DMA,
        pltpu.SemaphoreType.DMA,
    ],
)
def _sc_sweep(user_hbm, item_hbm, gutT, gitT, mutT, mitT, out_hbm,
              idx2, rowbuf, outbuf, s0, s1, s2, s3):
    wid = lax.axis_index("s") * _NC + lax.axis_index("c")
    tbl = wid // 8
    dbase = (wid % 8) * _TASKS_PER_W
    pltpu.sync_copy(user_hbm, idx2.at[pl.ds(0, _B)])
    pltpu.sync_copy(item_hbm, idx2.at[pl.ds(_B, _B)])
    ibase = (tbl % 2) * _B

    def task(k, carry):
        d = dbase + k
        for t, tab in enumerate((gutT, gitT, mutT, mitT)):
            @pl.when(tbl == t)
            def _():
                pltpu.async_copy(tab.at[d], rowbuf, s0).wait()

        def chunk(c, carry2):
            for j in range(_UNROLL):
                off = (c * _UNROLL + j) * _L
                ids = idx2[pl.ds(ibase + off, _L)]
                outbuf[pl.ds(off, _L)] = plsc.load_gather(rowbuf, [ids])
            return carry2

        lax.fori_loop(0, _B // (_L * _UNROLL), chunk, 0)
        pltpu.sync_copy(outbuf, out_hbm.at[wid * _TASKS_PER_W + k])
        return carry

    lax.fori_loop(0, _TASKS_PER_W, task, 0)


def _tc_body(act_ref, w1ut_ref, w1it_ref, b1_ref, w2t_ref, b2_ref,
             wpg_ref, wpm_ref, out_ref):
    gmf = act_ref[0:_D, :] * act_ref[_D:2 * _D, :]
    h = jnp.dot(w1ut_ref[...], act_ref[2 * _D:3 * _D, :],
                preferred_element_type=jnp.float32)
    h = h + jnp.dot(w1it_ref[...], act_ref[3 * _D:4 * _D, :],
                    preferred_element_type=jnp.float32)
    h = jnp.maximum(h + b1_ref[...], 0.0)
    m = jnp.dot(w2t_ref[...], h, preferred_element_type=jnp.float32)
    m = jnp.maximum(m + b2_ref[...], 0.0)
    out = jnp.dot(wpg_ref[...], gmf, preferred_element_type=jnp.float32)
    out = out + jnp.dot(wpm_ref[...], m, preferred_element_type=jnp.float32)
    out_ref[...] = out


def kernel(user, item, gmf_user_table, gmf_item_table,
           mlp_user_table, mlp_item_table, W1, b1, W2, b2, Wp):
    user = user.astype(jnp.int32)
    item = item.astype(jnp.int32)
    act = _sc_sweep(user, item,
                    gmf_user_table.T, gmf_item_table.T,
                    mlp_user_table.T, mlp_item_table.T)
    out = pl.pallas_call(
        _tc_body,
        out_shape=jax.ShapeDtypeStruct((1, _B), jnp.float32),
    )(act,
      W1[:_D].T, W1[_D:].T, b1.reshape(_D, 1),
      W2.T, b2.reshape(32, 1),
      Wp[:_D].reshape(1, _D), Wp[_D:].reshape(1, 32))
    return out.reshape(-1)


# speedup vs baseline: 3.3273x; 1.0887x over previous
import functools

import jax
import jax.numpy as jnp
from jax import lax
from jax.experimental import pallas as pl
from jax.experimental.pallas import tpu as pltpu
from jax.experimental.pallas import tpu_sc as plsc

_NC = 2
_NS = 16
_L = 16
_NW = _NC * _NS

_B = 4096
_H = _B // 2
_D = 64
_V = 100000
_TASKS_PER_W = (4 * _D) // _NW
_UNROLL = 4

_mesh = plsc.VectorSubcoreMesh(core_axis_name="c", subcore_axis_name="s")


@functools.partial(
    pl.kernel,
    out_type=jax.ShapeDtypeStruct((4 * _D, _H), jnp.float32),
    mesh=_mesh,
    compiler_params=pltpu.CompilerParams(needs_layout_passes=False),
    scratch_types=[
        pltpu.VMEM((2 * _B,), jnp.int32),
        pltpu.VMEM((_V,), jnp.float32),
        pltpu.VMEM((_H,), jnp.float32),
        pltpu.SemaphoreType.DMA,
    ],
)
def _sc_sweep(user_hbm, item_hbm, gutT, gitT, mutT, mitT, out_hbm,
              idx2, rowbuf, outbuf, s0):
    wid = lax.axis_index("s") * _NC + lax.axis_index("c")
    tbl = wid // 8
    dbase = (wid % 8) * _TASKS_PER_W
    pltpu.sync_copy(user_hbm, idx2.at[pl.ds(0, _B)])
    pltpu.sync_copy(item_hbm, idx2.at[pl.ds(_B, _B)])
    ibase = (tbl % 2) * _B

    def task(k, carry):
        d = dbase + k
        for t, tab in enumerate((gutT, gitT, mutT, mitT)):
            @pl.when(tbl == t)
            def _():
                pltpu.async_copy(tab.at[d], rowbuf, s0).wait()

        def chunk(c, carry2):
            for j in range(_UNROLL):
                c2 = c * _UNROLL + j
                ids_a = idx2[pl.ds(ibase + c2 * _L, _L)]
                ids_b = idx2[pl.ds(ibase + _H + c2 * _L, _L)]
                a = plsc.load_gather(rowbuf, [ids_a])
                b = plsc.load_gather(rowbuf, [ids_b])
                packed = plsc.pack(a, b, format=plsc.PackFormat.INTERLEAVED)
                outbuf[pl.ds(c2 * _L, _L)] = plsc.bitcast(packed, jnp.float32)
            return carry2

        lax.fori_loop(0, _H // (_L * _UNROLL), chunk, 0)
        pltpu.sync_copy(outbuf, out_hbm.at[wid * _TASKS_PER_W + k])
        return carry

    lax.fori_loop(0, _TASKS_PER_W, task, 0)


def _tc_body(act_ref, w1ub_ref, w1ib_ref, b1b_ref, w2b_ref, b2b_ref,
             wpgb_ref, wpmb_ref, out_ref):
    a = pltpu.bitcast(act_ref[...], jnp.bfloat16)
    gmf = (a[0:2 * _D, :].astype(jnp.float32)
           * a[2 * _D:4 * _D, :].astype(jnp.float32))
    h = jnp.dot(w1ub_ref[...], a[4 * _D:6 * _D, :],
                preferred_element_type=jnp.float32)
    h = h + jnp.dot(w1ib_ref[...], a[6 * _D:8 * _D, :],
                    preferred_element_type=jnp.float32)
    h = jnp.maximum(h + b1b_ref[...], 0.0)
    m = jnp.dot(w2b_ref[...], h, preferred_element_type=jnp.float32)
    m = jnp.maximum(m + b2b_ref[...], 0.0)
    out = jnp.dot(wpgb_ref[...], gmf, preferred_element_type=jnp.float32)
    out = out + jnp.dot(wpmb_ref[...], m, preferred_element_type=jnp.float32)
    out_ref[...] = out


def kernel(user, item, gmf_user_table, gmf_item_table,
           mlp_user_table, mlp_item_table, W1, b1, W2, b2, Wp):
    user = user.astype(jnp.int32)
    item = item.astype(jnp.int32)
    act = _sc_sweep(user, item,
                    gmf_user_table.T, gmf_item_table.T,
                    mlp_user_table.T, mlp_item_table.T)
    eye2 = jnp.eye(2, dtype=jnp.float32)
    w1ub = (W1[:_D].T[None, :, :, None] * eye2[:, None, None, :]
            ).reshape(2 * _D, 2 * _D).astype(jnp.bfloat16)
    w1ib = (W1[_D:].T[None, :, :, None] * eye2[:, None, None, :]
            ).reshape(2 * _D, 2 * _D).astype(jnp.bfloat16)
    b1b = jnp.tile(b1, 2).reshape(2 * _D, 1)
    w2b = (W2.T[None, :, None, :] * eye2[:, None, :, None]).reshape(_D, 2 * _D)
    b2b = jnp.tile(b2, 2).reshape(_D, 1)
    wpgb = (Wp[:_D, 0][None, :, None] * eye2[:, None, :]).reshape(2, 2 * _D)
    wpmb = (Wp[_D:, 0][None, None, :] * eye2[:, :, None]).reshape(2, _D)
    res = pl.pallas_call(
        _tc_body,
        out_shape=jax.ShapeDtypeStruct((2, _H), jnp.float32),
    )(act, w1ub, w1ib, b1b, w2b, b2b, wpgb, wpmb)
    return res.reshape(-1)


# speedup vs baseline: 3.3278x; 1.0001x over previous
import functools

import jax
import jax.numpy as jnp
from jax import lax
from jax.experimental import pallas as pl
from jax.experimental.pallas import tpu as pltpu
from jax.experimental.pallas import tpu_sc as plsc

_NC = 2
_NS = 16
_L = 16
_NW = _NC * _NS

_B = 4096
_H = _B // 2
_D = 64
_V = 100000
_TASKS_PER_W = (4 * _D) // _NW
_UNROLL = 4

_mesh = plsc.VectorSubcoreMesh(core_axis_name="c", subcore_axis_name="s")


@functools.partial(
    pl.kernel,
    out_type=jax.ShapeDtypeStruct((4 * _D, _H), jnp.float32),
    mesh=_mesh,
    compiler_params=pltpu.CompilerParams(needs_layout_passes=False),
    scratch_types=[
        pltpu.VMEM((2 * _B,), jnp.int32),
        pltpu.VMEM((_V,), jnp.float32),
        pltpu.VMEM((_H,), jnp.float32),
        pltpu.VMEM((_H,), jnp.float32),
        pltpu.SemaphoreType.DMA,
        pltpu.SemaphoreType.DMA,
        pltpu.SemaphoreType.DMA,
    ],
)
def _sc_sweep(user_hbm, item_hbm, gutT, gitT, mutT, mitT, out_hbm,
              idx2, rowbuf, outbuf0, outbuf1, s0, s1, s2):
    wid = lax.axis_index("s") * _NC + lax.axis_index("c")
    tbl = wid // 8
    dbase = (wid % 8) * _TASKS_PER_W
    pltpu.sync_copy(user_hbm, idx2.at[pl.ds(0, _B)])
    pltpu.sync_copy(item_hbm, idx2.at[pl.ds(_B, _B)])
    ibase = (tbl % 2) * _B

    def one_task(k, outbuf, wsem):
        d = dbase + k
        for t, tab in enumerate((gutT, gitT, mutT, mitT)):
            @pl.when(tbl == t)
            def _():
                pltpu.async_copy(tab.at[d], rowbuf, s0).wait()

        def chunk(c, carry2):
            for j in range(_UNROLL):
                c2 = c * _UNROLL + j
                ids_a = idx2[pl.ds(ibase + c2 * _L, _L)]
                ids_b = idx2[pl.ds(ibase + _H + c2 * _L, _L)]
                a = plsc.load_gather(rowbuf, [ids_a])
                b = plsc.load_gather(rowbuf, [ids_b])
                packed = plsc.pack(a, b, format=plsc.PackFormat.INTERLEAVED)
                outbuf[pl.ds(c2 * _L, _L)] = plsc.bitcast(packed, jnp.float32)
            return carry2

        lax.fori_loop(0, _H // (_L * _UNROLL), chunk, 0)
        return pltpu.async_copy(outbuf, out_hbm.at[wid * _TASKS_PER_W + k],
                                wsem)

    def task_pair(k2, carry):
        wa = one_task(k2 * 2, outbuf0, s1)
        wb = one_task(k2 * 2 + 1, outbuf1, s2)
        wa.wait()
        wb.wait()
        return carry

    lax.fori_loop(0, _TASKS_PER_W // 2, task_pair, 0)


def _tc_body(act_ref, w1ub_ref, w1ib_ref, b1b_ref, w2b_ref, b2b_ref,
             wpgb_ref, wpmb_ref, out_ref):
    a = pltpu.bitcast(act_ref[...], jnp.bfloat16)
    gmf = (a[0:2 * _D, :].astype(jnp.float32)
           * a[2 * _D:4 * _D, :].astype(jnp.float32))
    h = jnp.dot(w1ub_ref[...], a[4 * _D:6 * _D, :],
                preferred_element_type=jnp.float32)
    h = h + jnp.dot(w1ib_ref[...], a[6 * _D:8 * _D, :],
                    preferred_element_type=jnp.float32)
    h = jnp.maximum(h + b1b_ref[...], 0.0)
    m = jnp.dot(w2b_ref[...], h, preferred_element_type=jnp.float32)
    m = jnp.maximum(m + b2b_ref[...], 0.0)
    out = jnp.dot(wpgb_ref[...], gmf, preferred_element_type=jnp.float32)
    out = out + jnp.dot(wpmb_ref[...], m, preferred_element_type=jnp.float32)
    out_ref[...] = out


def kernel(user, item, gmf_user_table, gmf_item_table,
           mlp_user_table, mlp_item_table, W1, b1, W2, b2, Wp):
    user = user.astype(jnp.int32)
    item = item.astype(jnp.int32)
    act = _sc_sweep(user, item,
                    gmf_user_table.T, gmf_item_table.T,
                    mlp_user_table.T, mlp_item_table.T)
    eye2 = jnp.eye(2, dtype=jnp.float32)
    w1ub = (W1[:_D].T[None, :, :, None] * eye2[:, None, None, :]
            ).reshape(2 * _D, 2 * _D).astype(jnp.bfloat16)
    w1ib = (W1[_D:].T[None, :, :, None] * eye2[:, None, None, :]
            ).reshape(2 * _D, 2 * _D).astype(jnp.bfloat16)
    b1b = jnp.tile(b1, 2).reshape(2 * _D, 1)
    w2b = (W2.T[None, :, None, :] * eye2[:, None, :, None]).reshape(_D, 2 * _D)
    b2b = jnp.tile(b2, 2).reshape(_D, 1)
    wpgb = (Wp[:_D, 0][None, :, None] * eye2[:, None, :]).reshape(2, 2 * _D)
    wpmb = (Wp[_D:, 0][None, None, :] * eye2[:, :, None]).reshape(2, _D)
    res = pl.pallas_call(
        _tc_body,
        out_shape=jax.ShapeDtypeStruct((2, _H), jnp.float32),
    )(act, w1ub, w1ib, b1b, w2b, b2b, wpgb, wpmb)
    return res.reshape(-1)


# speedup vs baseline: 3.3290x; 1.0003x over previous
import functools

import jax
import jax.numpy as jnp
from jax import lax
from jax.experimental import pallas as pl
from jax.experimental.pallas import tpu as pltpu
from jax.experimental.pallas import tpu_sc as plsc

_NC = 2
_NS = 16
_L = 16
_NW = _NC * _NS

_B = 4096
_H = _B // 2
_D = 64
_V = 100000
_TASKS_PER_W = (4 * _D) // _NW
_UNROLL = 4

_mesh = plsc.VectorSubcoreMesh(core_axis_name="c", subcore_axis_name="s")


@functools.partial(
    pl.kernel,
    out_type=jax.ShapeDtypeStruct((4 * _D, _H), jnp.float32),
    mesh=_mesh,
    compiler_params=pltpu.CompilerParams(needs_layout_passes=False),
    scratch_types=[
        pltpu.VMEM((2 * _B,), jnp.int32),
        pltpu.VMEM((_V,), jnp.float32),
        pltpu.VMEM((_H,), jnp.float32),
        pltpu.VMEM((_H,), jnp.float32),
        pltpu.SemaphoreType.DMA,
        pltpu.SemaphoreType.DMA,
        pltpu.SemaphoreType.DMA,
    ],
)
def _sc_sweep(user_hbm, item_hbm, gutT, gitT, mutT, mitT, out_hbm,
              idx2, rowbuf, outbuf0, outbuf1, s0, s1, s2):
    wid = lax.axis_index("s") * _NC + lax.axis_index("c")
    tbl = wid // 8
    dbase = (wid % 8) * _TASKS_PER_W
    tabs = (gutT, gitT, mutT, mitT)

    def issue_row(d):
        for t, tab in enumerate(tabs):
            @pl.when(tbl == t)
            def _():
                pltpu.async_copy(tab.at[d], rowbuf, s0)

    def wait_row(d):
        pltpu.make_async_copy(gutT.at[d], rowbuf, s0).wait()

    issue_row(dbase)
    pltpu.sync_copy(user_hbm, idx2.at[pl.ds(0, _B)])
    pltpu.sync_copy(item_hbm, idx2.at[pl.ds(_B, _B)])
    ibase = (tbl % 2) * _B

    def one_task(k, outbuf, wsem):
        d = dbase + k
        wait_row(d)

        def chunk(c, carry2):
            for j in range(_UNROLL):
                c2 = c * _UNROLL + j
                ids_a = idx2[pl.ds(ibase + c2 * _L, _L)]
                ids_b = idx2[pl.ds(ibase + _H + c2 * _L, _L)]
                a = plsc.load_gather(rowbuf, [ids_a])
                b = plsc.load_gather(rowbuf, [ids_b])
                packed = plsc.pack(a, b, format=plsc.PackFormat.INTERLEAVED)
                outbuf[pl.ds(c2 * _L, _L)] = plsc.bitcast(packed, jnp.float32)
            return carry2

        lax.fori_loop(0, _H // (_L * _UNROLL), chunk, 0)

        @pl.when(k < _TASKS_PER_W - 1)
        def _():
            issue_row(d + 1)

        return pltpu.async_copy(outbuf, out_hbm.at[wid * _TASKS_PER_W + k],
                                wsem)

    def task_pair(k2, carry):
        wa = one_task(k2 * 2, outbuf0, s1)
        wb = one_task(k2 * 2 + 1, outbuf1, s2)
        wa.wait()
        wb.wait()
        return carry

    lax.fori_loop(0, _TASKS_PER_W // 2, task_pair, 0)


def _tc_body(act_ref, w1ub_ref, w1ib_ref, b1b_ref, w2b_ref, b2b_ref,
             wpgb_ref, wpmb_ref, out_ref):
    a = pltpu.bitcast(act_ref[...], jnp.bfloat16)
    gmf = (a[0:2 * _D, :].astype(jnp.float32)
           * a[2 * _D:4 * _D, :].astype(jnp.float32))
    h = jnp.dot(w1ub_ref[...], a[4 * _D:6 * _D, :],
                preferred_element_type=jnp.float32)
    h = h + jnp.dot(w1ib_ref[...], a[6 * _D:8 * _D, :],
                    preferred_element_type=jnp.float32)
    h = jnp.maximum(h + b1b_ref[...], 0.0)
    m = jnp.dot(w2b_ref[...], h, preferred_element_type=jnp.float32)
    m = jnp.maximum(m + b2b_ref[...], 0.0)
    out = jnp.dot(wpgb_ref[...], gmf, preferred_element_type=jnp.float32)
    out = out + jnp.dot(wpmb_ref[...], m, preferred_element_type=jnp.float32)
    out_ref[...] = out


def kernel(user, item, gmf_user_table, gmf_item_table,
           mlp_user_table, mlp_item_table, W1, b1, W2, b2, Wp):
    user = user.astype(jnp.int32)
    item = item.astype(jnp.int32)
    act = _sc_sweep(user, item,
                    gmf_user_table.T, gmf_item_table.T,
                    mlp_user_table.T, mlp_item_table.T)
    eye2 = jnp.eye(2, dtype=jnp.float32)
    w1ub = (W1[:_D].T[None, :, :, None] * eye2[:, None, None, :]
            ).reshape(2 * _D, 2 * _D).astype(jnp.bfloat16)
    w1ib = (W1[_D:].T[None, :, :, None] * eye2[:, None, None, :]
            ).reshape(2 * _D, 2 * _D).astype(jnp.bfloat16)
    b1b = jnp.tile(b1, 2).reshape(2 * _D, 1)
    w2b = (W2.T[None, :, None, :] * eye2[:, None, :, None]).reshape(_D, 2 * _D)
    b2b = jnp.tile(b2, 2).reshape(_D, 1)
    wpgb = (Wp[:_D, 0][None, :, None] * eye2[:, None, :]).reshape(2, 2 * _D)
    wpmb = (Wp[_D:, 0][None, None, :] * eye2[:, :, None]).reshape(2, _D)
    res = pl.pallas_call(
        _tc_body,
        out_shape=jax.ShapeDtypeStruct((2, _H), jnp.float32),
    )(act, w1ub, w1ib, b1b, w2b, b2b, wpgb, wpmb)
    return res.reshape(-1)
